# baseline (speedup 1.0000x reference)
import functools
import jax
import jax.numpy as jnp
from jax import lax
from jax.experimental import pallas as pl
from jax.experimental.pallas import tpu as pltpu
from jax.experimental.pallas import tpu_sc as plsc


VOCAB_PAD = 1024
WORDS = VOCAB_PAD // 4
LANES = 16
NC, NS = 2, 16
NW = NC * NS
GROUP = 16


def _sc_hist_body(idxt_hbm, out_hbm, idx_v, cnt_v, sem, *, rows_per_w, hist):
    wid = lax.axis_index("s") * NC + lax.axis_index("c")
    base = wid * rows_per_w

    stage = pltpu.async_copy(idxt_hbm.at[:, pl.ds(base, rows_per_w)],
                             idx_v, sem)

    zero16 = jnp.zeros((LANES,), jnp.int32)
    groups_per_row = WORDS // LANES

    @plsc.parallel_loop(0, rows_per_w, unroll=4)
    def _(r):
        for j in range(groups_per_row):
            cnt_v[r, pl.ds(j * LANES, LANES)] = zero16

    stage.wait()

    one = jnp.full((LANES,), 1, jnp.int32)
    lane = lax.iota(jnp.int32, LANES)
    ngroups = rows_per_w // GROUP
    rvecs = [lane + g * GROUP for g in range(ngroups)]
    half_rows = rows_per_w // 2
    halves = (range(ngroups // 2), range(ngroups // 2, ngroups))

    @plsc.parallel_loop(0, hist, unroll=2)
    def _(l):
        for g in halves[0]:
            vidx = idx_v[l, pl.ds(g * GROUP, GROUP)]
            val = one << ((vidx >> 8) << 3)
            plsc.addupdate_scatter(cnt_v, [rvecs[g], vidx & 255], val)

    wb = pltpu.async_copy(
        cnt_v.at[pl.ds(0, half_rows), :],
        out_hbm.at[pl.ds(base, half_rows), :], sem)

    @plsc.parallel_loop(0, hist, unroll=2)
    def _(l):
        for g in halves[1]:
            vidx = idx_v[l, pl.ds(g * GROUP, GROUP)]
            val = one << ((vidx >> 8) << 3)
            plsc.addupdate_scatter(cnt_v, [rvecs[g], vidx & 255], val)

    wb.wait()
    pltpu.sync_copy(cnt_v.at[pl.ds(half_rows, half_rows), :],
                    out_hbm.at[pl.ds(base + half_rows, half_rows), :])


def _sc_hist(inputs_t, *, b, hist):
    rows_per_w = b // NW
    mesh = plsc.VectorSubcoreMesh(core_axis_name="c", subcore_axis_name="s")
    body = functools.partial(_sc_hist_body, rows_per_w=rows_per_w, hist=hist)
    f = pl.kernel(
        body,
        out_type=jax.ShapeDtypeStruct((b, WORDS), jnp.int32),
        mesh=mesh,
        scratch_types=[
            pltpu.VMEM((hist, rows_per_w), jnp.int32),
            pltpu.VMEM((rows_per_w, WORDS), jnp.int32),
            pltpu.SemaphoreType.DMA,
        ],
        compiler_params=pltpu.CompilerParams(
            needs_layout_passes=False, use_tc_tiling_on_sc=True),
    )
    return f(inputs_t)


def _final_kernel(packed_ref, emb_ref, w_ref, b_ref, out_ref, m_ref,
                  *, inv_l, vocab):
    @pl.when(pl.program_id(0) == 0)
    def _():
        m_ref[...] = jnp.zeros_like(m_ref)
        m_ref[:vocab, :] = (lax.dot_general(
            emb_ref[...], w_ref[...],
            dimension_numbers=(((1,), (1,)), ((), ())),
            preferred_element_type=jnp.float32,
        ) + b_ref[...]) * inv_l

    packed = packed_ref[...]
    tb = packed.shape[0]
    zt = jnp.zeros((w_ref.shape[0], tb), jnp.float32)
    for p in range(4):
        plane = ((packed >> (8 * p)) & 255).astype(jnp.float32)
        zt = zt + lax.dot_general(
            m_ref[pl.ds(256 * p, 256), :], plane,
            dimension_numbers=(((0,), (1,)), ((), ())),
            preferred_element_type=jnp.float32,
        )
    zmax = jnp.max(zt, axis=0, keepdims=True)
    s = zt - zmax
    lse = jnp.log(jnp.sum(jnp.exp(s), axis=0, keepdims=True))
    out_ref[...] = s - lse


def kernel(inputs, emb, W, b):
    B, L = inputs.shape
    V, D = emb.shape
    C = W.shape[0]

    packed = _sc_hist(inputs.T, b=B, hist=L)

    TB = 2048
    b2 = b.reshape(1, C)
    out_t = pl.pallas_call(
        functools.partial(_final_kernel, inv_l=1.0 / L, vocab=V),
        grid=(B // TB,),
        in_specs=[
            pl.BlockSpec((TB, WORDS), lambda i: (i, 0)),
            pl.BlockSpec((V, D), lambda i: (0, 0)),
            pl.BlockSpec((C, D), lambda i: (0, 0)),
            pl.BlockSpec((1, C), lambda i: (0, 0)),
        ],
        out_specs=pl.BlockSpec((C, TB), lambda i: (0, i)),
        out_shape=jax.ShapeDtypeStruct((C, B), jnp.float32),
        scratch_shapes=[pltpu.VMEM((VOCAB_PAD, C), jnp.float32)],
    )(packed, emb, W, b2)

    return out_t.T


# speedup vs baseline: 1.0125x; 1.0125x over previous
import functools
import jax
import jax.numpy as jnp
from jax import lax
from jax.experimental import pallas as pl
from jax.experimental.pallas import tpu as pltpu
from jax.experimental.pallas import tpu_sc as plsc


VOCAB_PAD = 1024
WORDS = VOCAB_PAD // 4
LANES = 16
NC, NS = 2, 16
NW = NC * NS
GROUP = 16


def _sc_hist_body(idxt_hbm, out_hbm, idx_v, cnt_v, sem, *, rows_per_w, hist):
    wid = lax.axis_index("s") * NC + lax.axis_index("c")
    base = wid * rows_per_w

    stage = pltpu.async_copy(idxt_hbm.at[:, pl.ds(base, rows_per_w)],
                             idx_v, sem)

    zero16 = jnp.zeros((LANES,), jnp.int32)
    groups_per_row = WORDS // LANES

    @plsc.parallel_loop(0, rows_per_w, unroll=2)
    def _(r):
        for j in range(groups_per_row):
            cnt_v[r, pl.ds(j * LANES, LANES)] = zero16

    stage.wait()

    one = jnp.full((LANES,), 1, jnp.int32)
    lane = lax.iota(jnp.int32, LANES)
    ngroups = rows_per_w // GROUP
    rvecs = [lane + g * GROUP for g in range(ngroups)]
    half_rows = rows_per_w // 2
    halves = (range(ngroups // 2), range(ngroups // 2, ngroups))

    @plsc.parallel_loop(0, hist, unroll=2)
    def _(l):
        for g in halves[0]:
            vidx = idx_v[l, pl.ds(g * GROUP, GROUP)]
            val = one << ((vidx >> 8) << 3)
            plsc.addupdate_scatter(cnt_v, [rvecs[g], vidx & 255], val)

    wb = pltpu.async_copy(
        cnt_v.at[pl.ds(0, half_rows), :],
        out_hbm.at[pl.ds(base, half_rows), :], sem)

    @plsc.parallel_loop(0, hist, unroll=2)
    def _(l):
        for g in halves[1]:
            vidx = idx_v[l, pl.ds(g * GROUP, GROUP)]
            val = one << ((vidx >> 8) << 3)
            plsc.addupdate_scatter(cnt_v, [rvecs[g], vidx & 255], val)

    wb.wait()
    pltpu.sync_copy(cnt_v.at[pl.ds(half_rows, half_rows), :],
                    out_hbm.at[pl.ds(base + half_rows, half_rows), :])


def _sc_hist(inputs_t, *, b, hist):
    rows_per_w = b // NW
    mesh = plsc.VectorSubcoreMesh(core_axis_name="c", subcore_axis_name="s")
    body = functools.partial(_sc_hist_body, rows_per_w=rows_per_w, hist=hist)
    f = pl.kernel(
        body,
        out_type=jax.ShapeDtypeStruct((b, WORDS), jnp.int32),
        mesh=mesh,
        scratch_types=[
            pltpu.VMEM((hist, rows_per_w), jnp.int32),
            pltpu.VMEM((rows_per_w, WORDS), jnp.int32),
            pltpu.SemaphoreType.DMA,
        ],
        compiler_params=pltpu.CompilerParams(
            needs_layout_passes=False, use_tc_tiling_on_sc=True),
    )
    return f(inputs_t)


def _final_kernel(packed_ref, emb_ref, w_ref, b_ref, out_ref, m_ref,
                  *, inv_l, vocab):
    @pl.when(pl.program_id(0) == 0)
    def _():
        m_ref[...] = jnp.zeros_like(m_ref)
        m_ref[:vocab, :] = (lax.dot_general(
            emb_ref[...], w_ref[...],
            dimension_numbers=(((1,), (1,)), ((), ())),
            preferred_element_type=jnp.float32,
        ) + b_ref[...]) * inv_l

    packed = packed_ref[...]
    tb = packed.shape[0]
    zt = jnp.zeros((w_ref.shape[0], tb), jnp.float32)
    for p in range(4):
        plane = ((packed >> (8 * p)) & 255).astype(jnp.float32)
        zt = zt + lax.dot_general(
            m_ref[pl.ds(256 * p, 256), :], plane,
            dimension_numbers=(((0,), (1,)), ((), ())),
            preferred_element_type=jnp.float32,
        )
    zmax = jnp.max(zt, axis=0, keepdims=True)
    s = zt - zmax
    lse = jnp.log(jnp.sum(jnp.exp(s), axis=0, keepdims=True))
    out_ref[...] = s - lse


def kernel(inputs, emb, W, b):
    B, L = inputs.shape
    V, D = emb.shape
    C = W.shape[0]

    packed = _sc_hist(inputs.T, b=B, hist=L)

    TB = 2048
    b2 = b.reshape(1, C)
    out_t = pl.pallas_call(
        functools.partial(_final_kernel, inv_l=1.0 / L, vocab=V),
        grid=(B // TB,),
        in_specs=[
            pl.BlockSpec((TB, WORDS), lambda i: (i, 0)),
            pl.BlockSpec((V, D), lambda i: (0, 0)),
            pl.BlockSpec((C, D), lambda i: (0, 0)),
            pl.BlockSpec((1, C), lambda i: (0, 0)),
        ],
        out_specs=pl.BlockSpec((C, TB), lambda i: (0, i)),
        out_shape=jax.ShapeDtypeStruct((C, B), jnp.float32),
        scratch_shapes=[pltpu.VMEM((VOCAB_PAD, C), jnp.float32)],
    )(packed, emb, W, b2)

    return out_t.T
